# baseline (speedup 1.0000x reference)
import jax
import jax.numpy as jnp
from jax import lax
from jax.experimental import pallas as pl
from jax.experimental.pallas import tpu as pltpu
from jax.experimental.pallas import tpu_sc as plsc

MOM = 0.5
B = 16384
D = 128
N = 100000
NCL = 1000
NW = 32
BPW = B // NW
CORE = N // NW
EXT = 3136
NVI = B // 16
NVE = EXT // 16
CPYC = 256
FCAP = 3456
UPAD = 272
UNA = 8
TCR = 1024


def _gather_body(bank, ind_h, labs, fold_o, olab_o, idx_v, rows_v, lab_v, sem):
    wid = lax.axis_index("s") * 2 + lax.axis_index("c")
    base = wid * BPW
    pltpu.sync_copy(ind_h.at[pl.ds(base, BPW)], idx_v)
    pltpu.async_copy(bank.at[idx_v], rows_v, sem).wait()
    pltpu.sync_copy(rows_v, fold_o.at[pl.ds(base, BPW)])
    pltpu.async_copy(labs.at[idx_v], lab_v, sem).wait()
    pltpu.sync_copy(lab_v, olab_o.at[pl.ds(base, BPW)])


def _tc_body(feat, fold, cent, olab, fn2_o, nl_o, ch_o):
    i = pl.program_id(0)
    f = feat[...]
    fo = fold[...]
    fn = f / (jnp.sqrt(jnp.sum(f * f, axis=1, keepdims=True)) + 1e-10)
    fu = (1.0 - MOM) * fo + MOM * fn
    fn2 = fu / (jnp.sqrt(jnp.sum(fu * fu, axis=1, keepdims=True)) + 1e-10)
    fn2_o[...] = fn2
    sim = lax.dot_general(fn2, cent[...], (((1,), (1,)), ((), ())),
                          preferred_element_type=jnp.float32)
    mx = jnp.max(sim, axis=1, keepdims=True)
    ii = lax.broadcasted_iota(jnp.int32, sim.shape, 1)
    lbl = jnp.min(jnp.where(sim == mx, ii, jnp.int32(NCL)), axis=1)
    nl_o[0, 0, :] = lbl
    mism = jnp.sum((lbl != olab[0, 0, :]).astype(jnp.float32))
    prev = jnp.where(i == 0, 0.0, ch_o[0, 0])
    tot = prev + mism
    ch_o[0, 0] = jnp.where(i == pl.num_programs(0) - 1, tot / B, tot)


def _scatter_body(ind_h, fn2_h, nl_h, bank_h, labs_h, obank, olabs,
                  ind_v, nl_v, wref, lab_v, fi1, fx1, chunkb, upb, sem):
    wid = lax.axis_index("s") * 2 + lax.axis_index("c")
    base = wid * CORE
    start = pl.multiple_of(jnp.minimum(base - lax.rem(base, 8), N - EXT), 8)
    iota = lax.iota(jnp.int32, 16)
    pltpu.sync_copy(ind_h, ind_v)
    pltpu.sync_copy(nl_h, nl_v)
    pltpu.sync_copy(labs_h.at[pl.ds(start, EXT)], lab_v)

    def initw(k, _):
        for u in range(7):
            wref[pl.ds((k * 7 + u) * 16, 16)] = jnp.full((16,), -1, jnp.int32)
        return 0
    lax.fori_loop(0, NVE // 7, initw, 0)

    def initf(k, _):
        for u in range(8):
            fi1[pl.ds((k * 8 + u) * 16, 16)] = jnp.zeros((16,), jnp.int32)
        return 0
    lax.fori_loop(0, FCAP // 128, initf, 0)

    def passa(c, _):
        needacc = jnp.zeros((16,), jnp.int32)
        for u in range(UNA):
            j = c * UNA + u
            idx = ind_v[pl.ds(j * 16, 16)]
            loc = idx - start
            mask = (loc >= 0) & (loc < EXT)
            locc = jnp.clip(loc, 0, EXT - 1)
            iv = j * 16 + iota
            plsc.store_scatter(wref, [locc], iv, mask=mask)
            cur = plsc.load_gather(wref, [locc], mask=mask)
            needacc = needacc + (mask & (cur < iv)).astype(jnp.int32)

        @pl.when(plsc.cumsum(needacc)[15] > 0)
        def _():
            for u in range(UNA):
                j = c * UNA + u
                idx = ind_v[pl.ds(j * 16, 16)]
                loc = idx - start
                mask = (loc >= 0) & (loc < EXT)
                locc = jnp.clip(loc, 0, EXT - 1)
                iv = j * 16 + iota

                def fix(t, _2):
                    cur2 = plsc.load_gather(wref, [locc], mask=mask)
                    nm = mask & (cur2 < iv)
                    plsc.store_scatter(wref, [locc], iv, mask=nm)
                    return 0
                lax.fori_loop(0, 15, fix, 0)
        return 0
    lax.fori_loop(0, NVI // UNA, passa, 0)

    def scan(k0, carry):
        off, counts = carry
        for u in range(4):
            k = k0 * 4 + u
            pos = k * 16 + iota
            wv = wref[pl.ds(k * 16, 16)]
            has = wv >= 0
            wc = jnp.clip(wv, 0, B - 1)
            newv = plsc.load_gather(nl_v, [wc], mask=has)
            labcur = lab_v[pl.ds(k * 16, 16)]
            lab_v[pl.ds(k * 16, 16)] = jnp.where(has, newv, labcur)
            gi = has.astype(jnp.int32)
            cs = plsc.cumsum(gi)
            posn = jnp.clip(off + cs - 1, 0, FCAP - 1)
            plsc.store_scatter(fi1, [posn], wv, mask=has)
            plsc.store_scatter(fx1, [posn], start + pos, mask=has)
            cnt = cs[15]
            counts = counts + jnp.where(iota == (k >> 4), cnt, 0)
            off = off + cnt
        return (off, counts)
    _, cnt_v = lax.fori_loop(0, NVE // 4, scan,
                             (jnp.int32(0), jnp.zeros((16,), jnp.int32)))
    ecs_v = plsc.cumsum(cnt_v) - cnt_v
    pltpu.sync_copy(lab_v, olabs.at[pl.ds(start, EXT)])

    def do_chunk(c, size):
        s = start + c * CPYC
        pltpu.sync_copy(bank_h.at[pl.ds(s, size)], chunkb.at[pl.ds(0, size)])
        off_c = jnp.sum(jnp.where(iota == c, ecs_v, 0))
        cnt_c = jnp.sum(jnp.where(iota == c, cnt_v, 0))

        @pl.when(cnt_c > 0)
        def _():
            a_c = pl.multiple_of(off_c - lax.rem(off_c, 8), 8)
            lead = off_c - a_c
            pltpu.async_copy(fn2_h.at[fi1.at[pl.ds(a_c, UPAD)]], upb,
                             sem).wait()

            def apply(t, _):
                for u in range(4):
                    r = t * 4 + u
                    v = fx1[pl.ds(off_c + r, 16)]
                    loc = jnp.where(r < cnt_c, v[0] - s, jnp.int32(CPYC))
                    for j in range(8):
                        chunkb[loc, pl.ds(j * 16, 16)] = (
                            upb[lead + r, pl.ds(j * 16, 16)])
                return 0
            lax.fori_loop(0, (cnt_c + 3) // 4, apply, 0)
        pltpu.sync_copy(chunkb.at[pl.ds(0, size)], obank.at[pl.ds(s, size)])
        return 0

    lax.fori_loop(0, EXT // CPYC, lambda c, _: do_chunk(c, CPYC), 0)
    do_chunk(jnp.int32(EXT // CPYC), EXT - (EXT // CPYC) * CPYC)


def kernel(feature, ind, feature_bank, cluster_centroids, cluster_label_bank):
    ind32 = ind.astype(jnp.int32)
    mesh = plsc.VectorSubcoreMesh(core_axis_name="c", subcore_axis_name="s")

    fold, olab = pl.kernel(
        _gather_body,
        out_type=[jax.ShapeDtypeStruct((B, D), jnp.float32),
                  jax.ShapeDtypeStruct((B,), jnp.int32)],
        mesh=mesh,
        scratch_types=[pltpu.VMEM((BPW,), jnp.int32),
                       pltpu.VMEM((BPW, D), jnp.float32),
                       pltpu.VMEM((BPW,), jnp.int32),
                       pltpu.SemaphoreType.DMA],
    )(feature_bank, ind32, cluster_label_bank)

    fn2, nl3, ch = pl.pallas_call(
        _tc_body,
        out_shape=[jax.ShapeDtypeStruct((B, D), jnp.float32),
                   jax.ShapeDtypeStruct((B // TCR, 1, TCR), jnp.int32),
                   jax.ShapeDtypeStruct((1, 1), jnp.float32)],
        grid=(B // TCR,),
        in_specs=[pl.BlockSpec((TCR, D), lambda i: (i, 0)),
                  pl.BlockSpec((TCR, D), lambda i: (i, 0)),
                  pl.BlockSpec((NCL, D), lambda i: (0, 0)),
                  pl.BlockSpec((1, 1, TCR), lambda i: (i, 0, 0))],
        out_specs=[pl.BlockSpec((TCR, D), lambda i: (i, 0)),
                   pl.BlockSpec((1, 1, TCR), lambda i: (i, 0, 0)),
                   pl.BlockSpec(memory_space=pltpu.SMEM)],
    )(feature, fold, cluster_centroids, olab.reshape(B // TCR, 1, TCR))
    newlabel = nl3.reshape(B)

    new_bank, new_labels = pl.kernel(
        _scatter_body,
        out_type=[jax.ShapeDtypeStruct((N, D), jnp.float32),
                  jax.ShapeDtypeStruct((N,), jnp.int32)],
        mesh=mesh,
        scratch_types=[pltpu.VMEM((B,), jnp.int32),
                       pltpu.VMEM((B,), jnp.int32),
                       pltpu.VMEM((EXT,), jnp.int32),
                       pltpu.VMEM((EXT,), jnp.int32),
                       pltpu.VMEM((FCAP,), jnp.int32),
                       pltpu.VMEM((FCAP,), jnp.int32),
                       pltpu.VMEM((CPYC + 1, D), jnp.float32),
                       pltpu.VMEM((UPAD, D), jnp.float32),
                       pltpu.SemaphoreType.DMA],
        compiler_params=pltpu.CompilerParams(needs_layout_passes=False),
    )(ind32, fn2, newlabel, feature_bank, cluster_label_bank)

    return (ch.reshape(()), fn2, new_bank, new_labels)


# speedup vs baseline: 7.5527x; 7.5527x over previous
import jax
import jax.numpy as jnp
from jax import lax
from jax.experimental import pallas as pl
from jax.experimental.pallas import tpu as pltpu
from jax.experimental.pallas import tpu_sc as plsc

MOM = 0.5
B = 16384
D = 128
N = 100000
NCL = 1000
NW = 32
BPW = B // NW
CORE = N // NW
EXT = 3136
NVI = B // 16
NVE = EXT // 16
CPYC = 256
SUBB = 256
GB = 64
UNA = 8
TCR = 1024


def _gather_body(bank, ind_h, labs, fold_o, olab_o, idx_v, rows_v, lab_v, sem):
    wid = lax.axis_index("s") * 2 + lax.axis_index("c")
    base = wid * BPW
    pltpu.sync_copy(ind_h.at[pl.ds(base, BPW)], idx_v)
    pltpu.async_copy(bank.at[idx_v], rows_v, sem).wait()
    pltpu.sync_copy(rows_v, fold_o.at[pl.ds(base, BPW)])
    pltpu.async_copy(labs.at[idx_v], lab_v, sem).wait()
    pltpu.sync_copy(lab_v, olab_o.at[pl.ds(base, BPW)])


def _tc_body(feat, fold, cent, olab, fn2_o, nl_o, ch_o):
    i = pl.program_id(0)
    f = feat[...]
    fo = fold[...]
    fn = f / (jnp.sqrt(jnp.sum(f * f, axis=1, keepdims=True)) + 1e-10)
    fu = (1.0 - MOM) * fo + MOM * fn
    fn2 = fu / (jnp.sqrt(jnp.sum(fu * fu, axis=1, keepdims=True)) + 1e-10)
    fn2_o[...] = fn2
    sim = lax.dot_general(fn2, cent[...], (((1,), (1,)), ((), ())),
                          preferred_element_type=jnp.float32)
    mx = jnp.max(sim, axis=1, keepdims=True)
    ii = lax.broadcasted_iota(jnp.int32, sim.shape, 1)
    lbl = jnp.min(jnp.where(sim == mx, ii, jnp.int32(NCL)), axis=1)
    nl_o[0, 0, :] = lbl
    mism = jnp.sum((lbl != olab[0, 0, :]).astype(jnp.float32))
    prev = jnp.where(i == 0, 0.0, ch_o[0, 0])
    tot = prev + mism
    ch_o[0, 0] = jnp.where(i == pl.num_programs(0) - 1, tot / B, tot)


def _scatter_body(ind_h, fn2_h, nl_h, bank_h, labs_h, obank, olabs,
                  sca, nl_v, fw, idxb, lab_v, chunkb, upb, sem):
    wid = lax.axis_index("s") * 2 + lax.axis_index("c")
    base = wid * CORE
    start = pl.multiple_of(jnp.minimum(base - lax.rem(base, 8), N - EXT), 8)
    iota = lax.iota(jnp.int32, 16)
    pltpu.sync_copy(ind_h, sca.at[pl.ds(0, B)])
    pltpu.sync_copy(nl_h, nl_v.at[pl.ds(0, B)])
    pltpu.sync_copy(labs_h.at[pl.ds(start, EXT)], lab_v)

    def passa(c, off):
        for u in range(UNA):
            j = c * UNA + u
            idx = sca[pl.ds(j * 16, 16)]
            loc = idx - start
            mask = (loc >= 0) & (loc < EXT)
            packed = (loc << 14) | (j * 16 + iota)
            plsc.store_compressed(fw.at[pl.ds(off, 16)], packed, mask=mask)
            off = off + plsc.all_reduce_population_count(mask)[0]
        return off
    m = lax.fori_loop(0, NVI // UNA, passa, jnp.int32(0))

    def do_chunk(c, size):
        s = start + c * CPYC
        pltpu.sync_copy(bank_h.at[pl.ds(s, size)], chunkb.at[pl.ds(0, size)])
        lo = (c * CPYC) << 14
        hi = ((c * CPYC) + size) << 14

        def filt(t, o2):
            for u in range(4):
                tv = t * 4 + u
                p = fw[pl.ds(tv * 16, 16)]
                fm = (p >= lo) & (p < hi) & ((tv * 16 + iota) < m)
                plsc.store_compressed(sca.at[pl.ds(o2, 16)], p, mask=fm)
                o2 = o2 + plsc.all_reduce_population_count(fm)[0]
            return o2
        cnt = lax.fori_loop(0, (m + 63) // 64, filt, jnp.int32(0))

        @pl.when(cnt > 0)
        def _():
            def sb_loop(sb, _):
                sboff = sb * SUBB
                for t in range(SUBB // 16):
                    pv = sca[pl.ds(sboff + t * 16, 16)]
                    idxb[pl.ds(t * 16, 16)] = pv & (B - 1)
                sbcnt = jnp.minimum(cnt - sboff, SUBB)
                for g in range(SUBB // GB):
                    @pl.when(g * GB < sbcnt)
                    def _g():
                        pltpu.async_copy(
                            fn2_h.at[idxb.at[pl.ds(g * GB, GB)]],
                            upb.at[pl.ds(g * GB, GB)], sem).wait()

                def apply(r, _2):
                    pv = sca[pl.ds(sboff + r, 16)]
                    loce = pv[0] >> 14
                    loc = loce - c * CPYC
                    iv0 = pv[0] & (B - 1)
                    for jj in range(8):
                        chunkb[loc, pl.ds(jj * 16, 16)] = (
                            upb[r, pl.ds(jj * 16, 16)])
                    lane = loce & 15
                    basel = loce - lane
                    lv = nl_v[pl.ds(iv0, 16)]
                    cur = lab_v[pl.ds(basel, 16)]
                    lab_v[pl.ds(basel, 16)] = jnp.where(iota == lane,
                                                        lv[0], cur)
                    return 0
                lax.fori_loop(0, sbcnt, apply, 0)
                return 0
            lax.fori_loop(0, (cnt + SUBB - 1) // SUBB, sb_loop, 0)
        pltpu.sync_copy(chunkb.at[pl.ds(0, size)], obank.at[pl.ds(s, size)])
        return 0

    lax.fori_loop(0, EXT // CPYC, lambda c, _: do_chunk(c, CPYC), 0)
    do_chunk(jnp.int32(EXT // CPYC), EXT - (EXT // CPYC) * CPYC)
    pltpu.sync_copy(lab_v, olabs.at[pl.ds(start, EXT)])


def kernel(feature, ind, feature_bank, cluster_centroids, cluster_label_bank):
    ind32 = ind.astype(jnp.int32)
    mesh = plsc.VectorSubcoreMesh(core_axis_name="c", subcore_axis_name="s")

    fold, olab = pl.kernel(
        _gather_body,
        out_type=[jax.ShapeDtypeStruct((B, D), jnp.float32),
                  jax.ShapeDtypeStruct((B,), jnp.int32)],
        mesh=mesh,
        scratch_types=[pltpu.VMEM((BPW,), jnp.int32),
                       pltpu.VMEM((BPW, D), jnp.float32),
                       pltpu.VMEM((BPW,), jnp.int32),
                       pltpu.SemaphoreType.DMA],
    )(feature_bank, ind32, cluster_label_bank)

    fn2, nl3, ch = pl.pallas_call(
        _tc_body,
        out_shape=[jax.ShapeDtypeStruct((B, D), jnp.float32),
                   jax.ShapeDtypeStruct((B // TCR, 1, TCR), jnp.int32),
                   jax.ShapeDtypeStruct((1, 1), jnp.float32)],
        grid=(B // TCR,),
        in_specs=[pl.BlockSpec((TCR, D), lambda i: (i, 0)),
                  pl.BlockSpec((TCR, D), lambda i: (i, 0)),
                  pl.BlockSpec((NCL, D), lambda i: (0, 0)),
                  pl.BlockSpec((1, 1, TCR), lambda i: (i, 0, 0))],
        out_specs=[pl.BlockSpec((TCR, D), lambda i: (i, 0)),
                   pl.BlockSpec((1, 1, TCR), lambda i: (i, 0, 0)),
                   pl.BlockSpec(memory_space=pltpu.SMEM)],
    )(feature, fold, cluster_centroids, olab.reshape(B // TCR, 1, TCR))
    newlabel = nl3.reshape(B)

    new_bank, new_labels = pl.kernel(
        _scatter_body,
        out_type=[jax.ShapeDtypeStruct((N, D), jnp.float32),
                  jax.ShapeDtypeStruct((N,), jnp.int32)],
        mesh=mesh,
        scratch_types=[pltpu.VMEM((B + 64,), jnp.int32),
                       pltpu.VMEM((B + 16,), jnp.int32),
                       pltpu.VMEM((B + 64,), jnp.int32),
                       pltpu.VMEM((SUBB,), jnp.int32),
                       pltpu.VMEM((EXT,), jnp.int32),
                       pltpu.VMEM((CPYC, D), jnp.float32),
                       pltpu.VMEM((SUBB, D), jnp.float32),
                       pltpu.SemaphoreType.DMA],
        compiler_params=pltpu.CompilerParams(needs_layout_passes=False),
    )(ind32, fn2, newlabel, feature_bank, cluster_label_bank)

    return (ch.reshape(()), fn2, new_bank, new_labels)
